# Initial kernel scaffold; baseline (speedup 1.0000x reference)
import functools

import jax
import jax.numpy as jnp
from jax import lax
from jax.experimental import pallas as pl
from jax.experimental.pallas import tpu as pltpu
from jax.experimental.pallas import tpu_sc as plsc

N_NODES = 10000
N_EDGES = 320000
D_FEAT = 128

NC = 2
NS = 16
LANES = 16
NW = NC * NS
EPW = N_EDGES // NW
VECS = EPW // LANES


def _scores_tc_kernel(x_ref, w_ref, bias_ref, out_ref):
    res = lax.dot_general(
        x_ref[...], w_ref[...],
        dimension_numbers=(((1,), (1,)), ((), ())),
        preferred_element_type=jnp.float32,
    )
    out_ref[...] = res + bias_ref[...]


def _node_scores(x, W, b):
    w2 = W.reshape(2, D_FEAT)
    bias = jnp.stack([b[0], jnp.zeros((), jnp.float32)]).reshape(1, 2)
    return pl.pallas_call(
        _scores_tc_kernel,
        out_shape=jax.ShapeDtypeStruct((N_NODES, 2), jnp.float32),
    )(x, w2, bias)


def _edge_score_body(tab_hbm, ei_hbm, out_hbm, tab_v, src_v, dst_v, out_v):
    wid = lax.axis_index("s") * NC + lax.axis_index("c")
    base = wid * EPW
    pltpu.sync_copy(tab_hbm, tab_v)
    pltpu.sync_copy(ei_hbm.at[0, pl.ds(base, EPW)], src_v)
    pltpu.sync_copy(ei_hbm.at[1, pl.ds(base, EPW)], dst_v)

    def body(i, carry):
        off = pl.multiple_of(i * LANES, LANES)
        si = src_v[pl.ds(off, LANES)]
        di = dst_v[pl.ds(off, LANES)]
        a = plsc.load_gather(tab_v, [si * 2])
        c = plsc.load_gather(tab_v, [di * 2 + 1])
        z = a + c
        out_v[pl.ds(off, LANES)] = 1.0 / (1.0 + jnp.exp(-z))
        return carry

    lax.fori_loop(0, VECS, body, 0, unroll=4)
    pltpu.sync_copy(out_v, out_hbm.at[pl.ds(base, EPW)])


_edge_scores = functools.partial(
    pl.kernel,
    out_type=jax.ShapeDtypeStruct((N_EDGES,), jnp.float32),
    mesh=plsc.VectorSubcoreMesh(
        core_axis_name="c", subcore_axis_name="s", num_cores=NC,
        num_subcores=NS,
    ),
    scratch_types=[
        pltpu.VMEM((2 * N_NODES,), jnp.float32),
        pltpu.VMEM((EPW,), jnp.int32),
        pltpu.VMEM((EPW,), jnp.int32),
        pltpu.VMEM((EPW,), jnp.float32),
    ],
)(_edge_score_body)


def kernel(x, edge_index, W, b):
    tab = _node_scores(x, W, b).reshape(2 * N_NODES)
    return _edge_scores(tab, edge_index)


# speedup vs baseline: 26.2597x; 26.2597x over previous
import functools

import jax
import jax.numpy as jnp
from jax import lax
from jax.experimental import pallas as pl
from jax.experimental.pallas import tpu as pltpu
from jax.experimental.pallas import tpu_sc as plsc

N_NODES = 10000
N_EDGES = 320000
D_FEAT = 128

NC = 2
NS = 16
LANES = 16
NW = NC * NS
EPW = N_EDGES // NW
VECS = EPW // LANES


def _scores_tc_kernel(x_ref, w_ref, bias_ref, out_ref):
    res = lax.dot_general(
        x_ref[...], w_ref[...],
        dimension_numbers=(((1,), (1,)), ((), ())),
        preferred_element_type=jnp.float32,
    )
    out_ref[...] = res + bias_ref[...]


def _node_scores(x, W, b):
    w2 = W.reshape(2, D_FEAT)
    bias = jnp.stack([b[0], jnp.zeros((), jnp.float32)]).reshape(1, 2)
    return pl.pallas_call(
        _scores_tc_kernel,
        out_shape=jax.ShapeDtypeStruct((N_NODES, 2), jnp.float32),
    )(x, w2, bias)


def _edge_score_body(tab_hbm, src_hbm, dst_hbm, out_hbm, tab_v, src_v, dst_v,
                     out_v):
    wid = lax.axis_index("s") * NC + lax.axis_index("c")
    base = wid * EPW
    pltpu.sync_copy(tab_hbm, tab_v)
    pltpu.sync_copy(src_hbm.at[pl.ds(base, EPW)], src_v)
    pltpu.sync_copy(dst_hbm.at[pl.ds(base, EPW)], dst_v)

    def body(i, carry):
        off = pl.multiple_of(i * LANES, LANES)
        si = src_v[pl.ds(off, LANES)]
        di = dst_v[pl.ds(off, LANES)]
        a = plsc.load_gather(tab_v, [si * 2])
        c = plsc.load_gather(tab_v, [di * 2 + 1])
        z = a + c
        out_v[pl.ds(off, LANES)] = 1.0 / (1.0 + jnp.exp(-z))
        return carry

    lax.fori_loop(0, VECS, body, 0, unroll=4)
    pltpu.sync_copy(out_v, out_hbm.at[pl.ds(base, EPW)])


_edge_scores = functools.partial(
    pl.kernel,
    out_type=jax.ShapeDtypeStruct((N_EDGES,), jnp.float32),
    mesh=plsc.VectorSubcoreMesh(
        core_axis_name="c", subcore_axis_name="s", num_cores=NC,
        num_subcores=NS,
    ),
    scratch_types=[
        pltpu.VMEM((2 * N_NODES,), jnp.float32),
        pltpu.VMEM((EPW,), jnp.int32),
        pltpu.VMEM((EPW,), jnp.int32),
        pltpu.VMEM((EPW,), jnp.float32),
    ],
    compiler_params=pltpu.CompilerParams(needs_layout_passes=False),
)(_edge_score_body)


def kernel(x, edge_index, W, b):
    tab = _node_scores(x, W, b).reshape(2 * N_NODES)
    return _edge_scores(tab, edge_index[0], edge_index[1])


# speedup vs baseline: 43.9747x; 1.6746x over previous
import functools

import jax
import jax.numpy as jnp
from jax import lax
from jax.experimental import pallas as pl
from jax.experimental.pallas import tpu as pltpu
from jax.experimental.pallas import tpu_sc as plsc

N_NODES = 10000
N_EDGES = 320000
D_FEAT = 128

NC = 2
NS = 16
LANES = 16
NW = NC * NS
EPW = N_EDGES // NW
VECS = EPW // LANES


def _scores_tc_kernel(x_ref, w_ref, bias_ref, out_ref):
    res = lax.dot_general(
        x_ref[...], w_ref[...],
        dimension_numbers=(((1,), (1,)), ((), ())),
        preferred_element_type=jnp.float32,
    )
    out_ref[...] = res + bias_ref[...]


def _node_scores(x, W, b):
    w2 = W.reshape(2, D_FEAT)
    bias = jnp.stack([b[0], jnp.zeros((), jnp.float32)]).reshape(1, 2)
    return pl.pallas_call(
        _scores_tc_kernel,
        out_shape=jax.ShapeDtypeStruct((N_NODES, 2), jnp.float32),
    )(x, w2, bias)


def _edge_score_body(tab_hbm, ei_hbm, out_hbm, tab_v, src_v, dst_v, out_v,
                     sem):
    wid = lax.axis_index("s") * NC + lax.axis_index("c")
    base = wid * EPW
    cp1 = pltpu.async_copy(tab_hbm, tab_v, sem)
    cp2 = pltpu.async_copy(ei_hbm.at[pl.ds(base, EPW)], src_v, sem)
    cp3 = pltpu.async_copy(ei_hbm.at[pl.ds(N_EDGES + base, EPW)], dst_v, sem)
    cp1.wait()
    cp2.wait()
    cp3.wait()

    @plsc.parallel_loop(0, VECS, unroll=8)
    def _(i):
        off = pl.multiple_of(i * LANES, LANES)
        si = src_v[pl.ds(off, LANES)]
        di = dst_v[pl.ds(off, LANES)]
        a = plsc.load_gather(tab_v, [si * 2])
        c = plsc.load_gather(tab_v, [di * 2 + 1])
        z = a + c
        out_v[pl.ds(off, LANES)] = 1.0 / (1.0 + jnp.exp(-z))

    pltpu.sync_copy(out_v, out_hbm.at[pl.ds(base, EPW)])


_edge_scores = functools.partial(
    pl.kernel,
    out_type=jax.ShapeDtypeStruct((N_EDGES,), jnp.float32),
    mesh=plsc.VectorSubcoreMesh(
        core_axis_name="c", subcore_axis_name="s", num_cores=NC,
        num_subcores=NS,
    ),
    scratch_types=[
        pltpu.VMEM((2 * N_NODES,), jnp.float32),
        pltpu.VMEM((EPW,), jnp.int32),
        pltpu.VMEM((EPW,), jnp.int32),
        pltpu.VMEM((EPW,), jnp.float32),
        pltpu.SemaphoreType.DMA,
    ],
    compiler_params=pltpu.CompilerParams(needs_layout_passes=False),
)(_edge_score_body)


def kernel(x, edge_index, W, b):
    tab = _node_scores(x, W, b).reshape(2 * N_NODES)
    return _edge_scores(tab, edge_index.reshape(2 * N_EDGES))


# speedup vs baseline: 50.8518x; 1.1564x over previous
import functools

import jax
import jax.numpy as jnp
from jax import lax
from jax.experimental import pallas as pl
from jax.experimental.pallas import tpu as pltpu
from jax.experimental.pallas import tpu_sc as plsc

N_NODES = 10000
N_EDGES = 320000
D_FEAT = 128

NC = 2
NS = 16
LANES = 16
NW = NC * NS
EPW = N_EDGES // NW
VECS = EPW // LANES


ROW_BLK = 2048
N_BLKS = -(-N_NODES // ROW_BLK)


def _scores_tc_kernel(b_ref, x_ref, w_ref, out_ref):
    res = lax.dot_general(
        w_ref[...], x_ref[...],
        dimension_numbers=(((1,), (1,)), ((), ())),
        preferred_element_type=jnp.float32,
    )
    out_ref[0:1, :] = res[0:1, :] + b_ref[0, 0]
    out_ref[1:2, :] = res[1:2, :]


def _node_scores(x, W, b):
    w2 = W.reshape(2, D_FEAT)
    return pl.pallas_call(
        _scores_tc_kernel,
        grid=(N_BLKS,),
        in_specs=[
            pl.BlockSpec(memory_space=pltpu.SMEM),
            pl.BlockSpec((ROW_BLK, D_FEAT), lambda i: (i, 0)),
            pl.BlockSpec((2, D_FEAT), lambda i: (0, 0)),
        ],
        out_specs=pl.BlockSpec((2, ROW_BLK), lambda i: (0, i)),
        out_shape=jax.ShapeDtypeStruct((2, N_NODES), jnp.float32),
    )(b.reshape(1, 1), x, w2)


def _edge_score_body(tab_hbm, ei_hbm, out_hbm, tab_v, src_v, dst_v, out_v,
                     sem):
    wid = lax.axis_index("s") * NC + lax.axis_index("c")
    base = wid * EPW
    cp1 = pltpu.async_copy(tab_hbm, tab_v, sem)
    cp2 = pltpu.async_copy(ei_hbm.at[pl.ds(base, EPW)], src_v, sem)
    cp3 = pltpu.async_copy(ei_hbm.at[pl.ds(N_EDGES + base, EPW)], dst_v, sem)
    cp1.wait()
    cp2.wait()
    cp3.wait()

    @plsc.parallel_loop(0, VECS, unroll=8)
    def _(i):
        off = pl.multiple_of(i * LANES, LANES)
        si = src_v[pl.ds(off, LANES)]
        di = dst_v[pl.ds(off, LANES)]
        a = plsc.load_gather(tab_v, [si])
        c = plsc.load_gather(tab_v, [di + N_NODES])
        z = a + c
        out_v[pl.ds(off, LANES)] = 1.0 / (1.0 + jnp.exp(-z))

    pltpu.sync_copy(out_v, out_hbm.at[pl.ds(base, EPW)])


_edge_scores = functools.partial(
    pl.kernel,
    out_type=jax.ShapeDtypeStruct((N_EDGES,), jnp.float32),
    mesh=plsc.VectorSubcoreMesh(
        core_axis_name="c", subcore_axis_name="s", num_cores=NC,
        num_subcores=NS,
    ),
    scratch_types=[
        pltpu.VMEM((2 * N_NODES,), jnp.float32),
        pltpu.VMEM((EPW,), jnp.int32),
        pltpu.VMEM((EPW,), jnp.int32),
        pltpu.VMEM((EPW,), jnp.float32),
        pltpu.SemaphoreType.DMA,
    ],
    compiler_params=pltpu.CompilerParams(needs_layout_passes=False),
)(_edge_score_body)


def kernel(x, edge_index, W, b):
    tab = _node_scores(x, W, b).reshape(2 * N_NODES)
    return _edge_scores(tab, edge_index.reshape(2 * N_EDGES))
